# baseline (speedup 1.0000x reference)
import functools

import jax
import jax.numpy as jnp
from jax import lax
from jax.experimental import pallas as pl
from jax.experimental.pallas import tpu as pltpu
from jax.experimental.pallas import tpu_sc as plsc

_NC = 2
_NS = 16
_NW = _NC * _NS
_CH = 128
_L = 16
_NB = 3


def _linear(x, W, b):
    n, d_in = x.shape
    d_out = W.shape[0]
    blk = 2000

    def body(x_ref, wt_ref, b_ref, h_ref):
        h_ref[...] = (
            jnp.dot(x_ref[...], wt_ref[...], preferred_element_type=jnp.float32)
            + b_ref[...]
        )

    return pl.pallas_call(
        body,
        grid=(n // blk,),
        in_specs=[
            pl.BlockSpec((blk, d_in), lambda i: (i, 0)),
            pl.BlockSpec((d_in, d_out), lambda i: (0, 0)),
            pl.BlockSpec((1, d_out), lambda i: (0, 0)),
        ],
        out_specs=pl.BlockSpec((blk, d_out), lambda i: (i, 0)),
        out_shape=jax.ShapeDtypeStruct((n, d_out), jnp.float32),
    )(x, W.T, b[None, :])


def _combine(partials):
    nc, n, d = partials.shape
    blk = 2000

    def body(p_ref, o_ref):
        o_ref[...] = p_ref[0] + p_ref[1]

    return pl.pallas_call(
        body,
        grid=(n // blk,),
        in_specs=[pl.BlockSpec((nc, blk, d), lambda i: (0, i, 0))],
        out_specs=pl.BlockSpec((blk, d), lambda i: (i, 0)),
        out_shape=jax.ShapeDtypeStruct((n, d), jnp.float32),
    )(partials)


def _sc_aggregate(h, src, dst, w, zeros):
    n, d = h.shape
    e = src.shape[0]
    nch = e // _CH
    cpt = nch // _NW
    assert cpt % _NB == 0 and cpt >= 2 * _NB
    rows_per_tile = (n // _NS) // 8 * 8
    tail_rows = n - _NS * rows_per_tile
    assert tail_rows % 8 == 0
    mesh = plsc.VectorSubcoreMesh(core_axis_name="c", subcore_axis_name="s")

    @functools.partial(
        pl.kernel,
        out_type=jax.ShapeDtypeStruct((_NC, n, d), jnp.float32),
        mesh=mesh,
        compiler_params=pltpu.CompilerParams(needs_layout_passes=False),
        scratch_types=[
            [pltpu.VMEM((_CH,), jnp.int32)] * _NB,
            [pltpu.VMEM((_CH,), jnp.int32)] * _NB,
            [pltpu.VMEM((_CH,), jnp.float32)] * _NB,
            [pltpu.VMEM((_CH, d), jnp.float32)] * _NB,
            pltpu.VMEM_SHARED((n, d), jnp.float32),
            [pltpu.SemaphoreType.DMA] * _NB,
            [pltpu.SemaphoreType.DMA] * _NB,
            [pltpu.SemaphoreType.DMA] * _NB,
        ],
    )
    def agg(h_hbm, src_hbm, dst_hbm, w_hbm, z_hbm, out_hbm,
            srcs, dsts, ws, rows, accum, si, sg, ss):
        cid = lax.axis_index("c")
        sid = lax.axis_index("s")
        wid = cid * _NS + sid
        e0 = wid * cpt * _CH

        def idx_issue(j, u):
            off = e0 + j * _CH
            pltpu.async_copy(src_hbm.at[pl.ds(off, _CH)], srcs[u], si[u])
            pltpu.async_copy(dst_hbm.at[pl.ds(off, _CH)], dsts[u], si[u])
            pltpu.async_copy(w_hbm.at[pl.ds(off, _CH)], ws[u], si[u])

        def idx_wait(u):
            pltpu.make_async_copy(src_hbm.at[pl.ds(0, _CH)], srcs[u],
                                  si[u]).wait()
            pltpu.make_async_copy(dst_hbm.at[pl.ds(0, _CH)], dsts[u],
                                  si[u]).wait()
            pltpu.make_async_copy(w_hbm.at[pl.ds(0, _CH)], ws[u],
                                  si[u]).wait()

        def gather_issue(u):
            pltpu.async_copy(h_hbm.at[srcs[u]], rows[u], sg[u])

        def gather_wait(u):
            pltpu.make_async_copy(h_hbm.at[srcs[u]], rows[u], sg[u]).wait()

        def scatter_issue(u):
            pltpu.async_copy(rows[u], accum.at[dsts[u]], ss[u], add=True)

        def scatter_wait(u):
            pltpu.make_async_copy(rows[u], accum.at[dsts[u]], ss[u]).wait()

        idx_issue(0, 0)
        idx_issue(1, 1)

        r0 = sid * rows_per_tile
        pltpu.sync_copy(z_hbm.at[pl.ds(r0, rows_per_tile)],
                        accum.at[pl.ds(r0, rows_per_tile)])
        if tail_rows:
            @pl.when(sid == _NS - 1)
            def _zero_tail():
                t0 = _NS * rows_per_tile
                pltpu.sync_copy(z_hbm.at[pl.ds(t0, tail_rows)],
                                accum.at[pl.ds(t0, tail_rows)])
        plsc.subcore_barrier()

        idx_wait(0)
        gather_issue(0)

        nt = cpt // _NB

        @pl.loop(0, nt)
        def _block(t):
            for u in range(_NB):
                j = _NB * t + u
                up = (u + 1) % _NB
                um = (u + 2) % _NB
                gather_wait(u)
                if u == _NB - 1:
                    @pl.when(t < nt - 1)
                    def _g_next():
                        idx_wait(up)
                        gather_issue(up)
                else:
                    idx_wait(up)
                    gather_issue(up)

                def _scale(i, c2):
                    wv = plsc.load_gather(ws[u],
                                          [jnp.full((_L,), i, jnp.int32)])
                    for f in range(d // _L):
                        sl = (i, pl.ds(f * _L, _L))
                        rows[u][sl] = rows[u][sl] * wv
                    return c2

                lax.fori_loop(0, _CH, _scale, 0)

                scatter_issue(u)

                if u == 0:
                    @pl.when(t > 0)
                    def _drain0():
                        scatter_wait(um)
                    idx_issue(j + 2, um)
                else:
                    scatter_wait(um)

                    @pl.when(t < nt - 1)
                    def _i_next():
                        idx_issue(j + 2, um)

        scatter_wait((cpt - 1) % _NB)
        plsc.subcore_barrier()

        pltpu.sync_copy(accum.at[pl.ds(r0, rows_per_tile)],
                        out_hbm.at[cid, pl.ds(r0, rows_per_tile)])
        if tail_rows:
            @pl.when(sid == _NS - 1)
            def _write_tail():
                t0 = _NS * rows_per_tile
                pltpu.sync_copy(accum.at[pl.ds(t0, tail_rows)],
                                out_hbm.at[cid, pl.ds(t0, tail_rows)])

    return agg(h, src, dst, w, zeros)


def kernel(x, edge_index, edge_weight, W, b):
    h = _linear(x, W, b)
    zeros = jnp.zeros_like(h)

    e = edge_weight.shape[0]
    quantum = _NW * _CH * _NB
    ep = -(-e // quantum) * quantum
    pad = ep - e
    src = jnp.concatenate([edge_index[0], jnp.zeros((pad,), jnp.int32)])
    dst = jnp.concatenate([edge_index[1], jnp.zeros((pad,), jnp.int32)])
    w = jnp.concatenate([edge_weight, jnp.zeros((pad,), jnp.float32)])

    partials = _sc_aggregate(h, src, dst, w, zeros)
    return _combine(partials)


# speedup vs baseline: 2.4046x; 2.4046x over previous
import functools

import jax
import jax.numpy as jnp
from jax import lax
from jax.experimental import pallas as pl
from jax.experimental.pallas import tpu as pltpu
from jax.experimental.pallas import tpu_sc as plsc

_NC = 2
_NS = 16
_NW = _NC * _NS
_CH = 128
_L = 16


def _linear(x, W, b):
    n, d_in = x.shape
    d_out = W.shape[0]
    blk = 2000

    def body(x_ref, wt_ref, b_ref, h_ref):
        h_ref[...] = (
            jnp.dot(x_ref[...], wt_ref[...], preferred_element_type=jnp.float32)
            + b_ref[...]
        )

    return pl.pallas_call(
        body,
        grid=(n // blk,),
        in_specs=[
            pl.BlockSpec((blk, d_in), lambda i: (i, 0)),
            pl.BlockSpec((d_in, d_out), lambda i: (0, 0)),
            pl.BlockSpec((1, d_out), lambda i: (0, 0)),
        ],
        out_specs=pl.BlockSpec((blk, d_out), lambda i: (i, 0)),
        out_shape=jax.ShapeDtypeStruct((n, d_out), jnp.float32),
    )(x, W.T, b[None, :])


def _combine(partials):
    nc, n, d = partials.shape
    blk = 2000

    def body(p_ref, o_ref):
        o_ref[...] = p_ref[0] + p_ref[1]

    return pl.pallas_call(
        body,
        grid=(n // blk,),
        in_specs=[pl.BlockSpec((nc, blk, d), lambda i: (0, i, 0))],
        out_specs=pl.BlockSpec((blk, d), lambda i: (i, 0)),
        out_shape=jax.ShapeDtypeStruct((n, d), jnp.float32),
    )(partials)


def _sc_aggregate(h, packed, zeros):
    n, d = h.shape
    n_chunks = packed.shape[0]
    base_trips = n_chunks // _NW
    extra = n_chunks % _NW
    rows_per_tile = (n // _NS) // 8 * 8
    tail_rows = n - _NS * rows_per_tile
    assert tail_rows % 8 == 0
    mesh = plsc.VectorSubcoreMesh(core_axis_name="c", subcore_axis_name="s")

    @functools.partial(
        pl.kernel,
        out_type=jax.ShapeDtypeStruct((_NC, n, d), jnp.float32),
        mesh=mesh,
        compiler_params=pltpu.CompilerParams(needs_layout_passes=False),
        scratch_types=[
            pltpu.VMEM((3, _CH), jnp.int32),
            pltpu.VMEM((_CH, d), jnp.float32),
            pltpu.VMEM_SHARED((n, d), jnp.float32),
            pltpu.SemaphoreType.DMA,
        ],
    )
    def agg(h_hbm, p_hbm, z_hbm, out_hbm, ibuf, rows_v, accum, sem):
        cid = lax.axis_index("c")
        sid = lax.axis_index("s")
        wid = cid * _NS + sid

        r0 = sid * rows_per_tile
        pltpu.sync_copy(z_hbm.at[pl.ds(r0, rows_per_tile)],
                        accum.at[pl.ds(r0, rows_per_tile)])
        if tail_rows:
            @pl.when(sid == _NS - 1)
            def _zero_tail():
                t0 = _NS * rows_per_tile
                pltpu.sync_copy(z_hbm.at[pl.ds(t0, tail_rows)],
                                accum.at[pl.ds(t0, tail_rows)])
        plsc.subcore_barrier()

        ntrips = base_trips + jnp.where(wid < extra, 1, 0)

        def body(j, carry):
            c = wid + _NW * j
            pltpu.sync_copy(p_hbm.at[c], ibuf)
            pltpu.async_copy(h_hbm.at[ibuf.at[0]], rows_v, sem).wait()

            @plsc.parallel_loop(0, _CH, unroll=4)
            def _scale(i):
                wv = plsc.bitcast(
                    plsc.load_gather(
                        ibuf,
                        [jnp.full((_L,), 2, jnp.int32),
                         jnp.full((_L,), i, jnp.int32)]),
                    jnp.float32)
                for f in range(d // _L):
                    sl = (i, pl.ds(f * _L, _L))
                    rows_v[sl] = rows_v[sl] * wv

            pltpu.sync_copy(rows_v, accum.at[ibuf.at[1]], add=True)
            return carry

        lax.fori_loop(0, ntrips, body, 0)

        plsc.subcore_barrier()
        pltpu.sync_copy(accum.at[pl.ds(r0, rows_per_tile)],
                        out_hbm.at[cid, pl.ds(r0, rows_per_tile)])
        if tail_rows:
            @pl.when(sid == _NS - 1)
            def _write_tail():
                t0 = _NS * rows_per_tile
                pltpu.sync_copy(accum.at[pl.ds(t0, tail_rows)],
                                out_hbm.at[cid, pl.ds(t0, tail_rows)])

    return agg(h, packed, zeros)


def kernel(x, edge_index, edge_weight, W, b):
    h = _linear(x, W, b)
    zeros = jnp.zeros_like(h)
    wbits = lax.bitcast_convert_type(edge_weight, jnp.int32)
    packed = jnp.stack([
        edge_index[0].reshape(-1, _CH),
        edge_index[1].reshape(-1, _CH),
        wbits.reshape(-1, _CH),
    ], axis=1)
    partials = _sc_aggregate(h, packed, zeros)
    return _combine(partials)
